# Initial kernel scaffold; baseline (speedup 1.0000x reference)
import jax
import jax.numpy as jnp
from jax.experimental import pallas as pl


def kernel(token_logits_pitch, token_logits_duration, token_logits_velocity, log_lambda_pitch, log_lambda_duration, log_lambda_velocity, x_t_pitch, x_t_duration, x_t_velocity, x_0_pitch, x_0_duration, x_0_velocity, x_1_pitch, x_1_duration, x_1_velocity, t):
    raise NotImplementedError("write your pallas kernel here")


# speedup vs baseline: 2.6609x; 2.6609x over previous
import jax
import jax.numpy as jnp
from jax import lax
from jax.experimental import pallas as pl

_B, _S = 4, 8192
_T = _B * _S
_TB = 512
_NB = _T // _TB
_EPS = 1e-8
_NEG = float(jnp.finfo(jnp.float32).min)


def _softplus(x):
    return jnp.maximum(x, 0.0) + jnp.log1p(jnp.exp(-jnp.abs(x)))


def _attr_loss(l, xt_row, x0_row, x1_row, ll_row, tl_row, disallow_zero):
    TB, V = l.shape
    sm = (xt_row == x0_row) & (x1_row != xt_row)
    coef_row = jnp.where(sm, tl_row, 0.0)
    lam_row = _softplus(ll_row)

    xt = jnp.transpose(xt_row)
    x1 = jnp.transpose(x1_row)
    iota = lax.broadcasted_iota(jnp.int32, (TB, V), 1)
    maskm = iota == xt
    if disallow_zero:
        maskm = maskm | (iota == 0)
    lm = jnp.where(maskm, _NEG, l)
    m = jnp.max(lm, axis=1, keepdims=True)
    p = jnp.exp(lm - m)
    z = jnp.sum(p, axis=1, keepdims=True)
    p1 = jnp.sum(jnp.where(iota == x1, p, 0.0), axis=1, keepdims=True)
    px1_row = jnp.transpose(p1 / z)
    tok = lam_row - coef_row * jnp.log(lam_row * px1_row + _EPS)
    return jnp.sum(tok)


def _loss_kernel(lp_ref, ld_ref, lv_ref,
                 xtp_ref, x0p_ref, x1p_ref, llp_ref,
                 xtd_ref, x0d_ref, x1d_ref, lld_ref,
                 xtv_ref, x0v_ref, x1v_ref, llv_ref,
                 tl_ref, out_ref):
    i = pl.program_id(0)

    @pl.when(i == 0)
    def _():
        out_ref[...] = jnp.zeros_like(out_ref)

    tl_row = tl_ref[0]
    acc = _attr_loss(lp_ref[...], xtp_ref[0], x0p_ref[0], x1p_ref[0],
                     llp_ref[0], tl_row, False)
    acc += _attr_loss(ld_ref[...], xtd_ref[0], x0d_ref[0], x1d_ref[0],
                      lld_ref[0], tl_row, True)
    acc += _attr_loss(lv_ref[...], xtv_ref[0], x0v_ref[0], x1v_ref[0],
                      llv_ref[0], tl_row, True)
    out_ref[...] += jnp.full((1, 1), acc)


def kernel(token_logits_pitch, token_logits_duration, token_logits_velocity,
           log_lambda_pitch, log_lambda_duration, log_lambda_velocity,
           x_t_pitch, x_t_duration, x_t_velocity,
           x_0_pitch, x_0_duration, x_0_velocity,
           x_1_pitch, x_1_duration, x_1_velocity,
           t):
    tl = 1.0 / jnp.clip(1.0 - t, 1e-8, None)
    tl_full = jnp.broadcast_to(tl[:, None], (_B, _S))

    def meta(x):
        return x.reshape(_NB, 1, _TB)

    args = [
        token_logits_pitch.reshape(_T, 128),
        token_logits_duration.reshape(_T, 512),
        token_logits_velocity.reshape(_T, 128),
        meta(x_t_pitch), meta(x_0_pitch), meta(x_1_pitch), meta(log_lambda_pitch),
        meta(x_t_duration), meta(x_0_duration), meta(x_1_duration), meta(log_lambda_duration),
        meta(x_t_velocity), meta(x_0_velocity), meta(x_1_velocity), meta(log_lambda_velocity),
        meta(tl_full),
    ]

    def logits_spec(v):
        return pl.BlockSpec((_TB, v), lambda i: (i, 0))

    meta_spec = pl.BlockSpec((1, 1, _TB), lambda i: (i, 0, 0))

    out = pl.pallas_call(
        _loss_kernel,
        grid=(_NB,),
        in_specs=[logits_spec(128), logits_spec(512), logits_spec(128)]
                 + [meta_spec] * 13,
        out_specs=pl.BlockSpec((1, 1), lambda i: (0, 0)),
        out_shape=jax.ShapeDtypeStruct((1, 1), jnp.float32),
    )(*args)
    return out[0, 0] / _T


# speedup vs baseline: 3.0771x; 1.1564x over previous
import jax
import jax.numpy as jnp
from jax import lax
from jax.experimental import pallas as pl

_B, _S = 4, 8192
_T = _B * _S
_TB = 1024
_NB = _T // _TB
_EPS = 1e-8
_NEG = float(jnp.finfo(jnp.float32).min)


def _softplus(x):
    return jnp.maximum(x, 0.0) + jnp.log1p(jnp.exp(-jnp.abs(x)))


def _attr_loss(l, xt_row, x0_row, x1_row, ll_row, tl_row, disallow_zero):
    TB, V = l.shape
    sm = (xt_row == x0_row) & (x1_row != xt_row)
    coef_row = jnp.where(sm, tl_row, 0.0)
    lam_row = _softplus(ll_row)

    xt = jnp.transpose(xt_row)
    x1 = jnp.transpose(x1_row)
    iota = lax.broadcasted_iota(jnp.int32, (TB, V), 1)
    maskm = iota == xt
    if disallow_zero:
        maskm = maskm | (iota == 0)
    lm = jnp.where(maskm, _NEG, l)
    m = jnp.max(lm, axis=1, keepdims=True)
    p = jnp.exp(lm - m)
    z = jnp.sum(p, axis=1, keepdims=True)
    p1 = jnp.sum(jnp.where(iota == x1, p, 0.0), axis=1, keepdims=True)
    px1_row = jnp.transpose(p1 / z)
    tok = lam_row - coef_row * jnp.log(lam_row * px1_row + _EPS)
    return jnp.sum(tok)


def _loss_kernel(lp_ref, ld_ref, lv_ref,
                 xtp_ref, x0p_ref, x1p_ref, llp_ref,
                 xtd_ref, x0d_ref, x1d_ref, lld_ref,
                 xtv_ref, x0v_ref, x1v_ref, llv_ref,
                 tl_ref, out_ref):
    i = pl.program_id(0)

    @pl.when(i == 0)
    def _():
        out_ref[...] = jnp.zeros_like(out_ref)

    tl_row = tl_ref[0]
    acc = _attr_loss(lp_ref[...], xtp_ref[0], x0p_ref[0], x1p_ref[0],
                     llp_ref[0], tl_row, False)
    acc += _attr_loss(ld_ref[...], xtd_ref[0], x0d_ref[0], x1d_ref[0],
                      lld_ref[0], tl_row, True)
    acc += _attr_loss(lv_ref[...], xtv_ref[0], x0v_ref[0], x1v_ref[0],
                      llv_ref[0], tl_row, True)
    out_ref[...] += jnp.full((1, 1), acc)


def kernel(token_logits_pitch, token_logits_duration, token_logits_velocity,
           log_lambda_pitch, log_lambda_duration, log_lambda_velocity,
           x_t_pitch, x_t_duration, x_t_velocity,
           x_0_pitch, x_0_duration, x_0_velocity,
           x_1_pitch, x_1_duration, x_1_velocity,
           t):
    tl = 1.0 / jnp.clip(1.0 - t, 1e-8, None)
    tl_full = jnp.broadcast_to(tl[:, None], (_B, _S))

    def meta(x):
        return x.reshape(_NB, 1, _TB)

    args = [
        token_logits_pitch.reshape(_T, 128),
        token_logits_duration.reshape(_T, 512),
        token_logits_velocity.reshape(_T, 128),
        meta(x_t_pitch), meta(x_0_pitch), meta(x_1_pitch), meta(log_lambda_pitch),
        meta(x_t_duration), meta(x_0_duration), meta(x_1_duration), meta(log_lambda_duration),
        meta(x_t_velocity), meta(x_0_velocity), meta(x_1_velocity), meta(log_lambda_velocity),
        meta(tl_full),
    ]

    def logits_spec(v):
        return pl.BlockSpec((_TB, v), lambda i: (i, 0))

    meta_spec = pl.BlockSpec((1, 1, _TB), lambda i: (i, 0, 0))

    out = pl.pallas_call(
        _loss_kernel,
        grid=(_NB,),
        in_specs=[logits_spec(128), logits_spec(512), logits_spec(128)]
                 + [meta_spec] * 13,
        out_specs=pl.BlockSpec((1, 1), lambda i: (0, 0)),
        out_shape=jax.ShapeDtypeStruct((1, 1), jnp.float32),
    )(*args)
    return out[0, 0] / _T


# speedup vs baseline: 3.2007x; 1.0402x over previous
import jax
import jax.numpy as jnp
from jax import lax
from jax.experimental import pallas as pl

_B, _S = 4, 8192
_T = _B * _S
_TB = 2048
_NB = _T // _TB
_EPS = 1e-8
_NEG = float(jnp.finfo(jnp.float32).min)


def _softplus(x):
    return jnp.maximum(x, 0.0) + jnp.log1p(jnp.exp(-jnp.abs(x)))


def _attr_loss(l, xt_row, x0_row, x1_row, ll_row, tl_row, disallow_zero):
    TB, V = l.shape
    sm = (xt_row == x0_row) & (x1_row != xt_row)
    coef_row = jnp.where(sm, tl_row, 0.0)
    lam_row = _softplus(ll_row)

    xt = jnp.transpose(xt_row)
    x1 = jnp.transpose(x1_row)
    iota = lax.broadcasted_iota(jnp.int32, (TB, V), 1)
    maskm = iota == xt
    if disallow_zero:
        maskm = maskm | (iota == 0)
    lm = jnp.where(maskm, _NEG, l)
    m = jnp.max(lm, axis=1, keepdims=True)
    p = jnp.exp(lm - m)
    z = jnp.sum(p, axis=1, keepdims=True)
    p1 = jnp.sum(jnp.where(iota == x1, p, 0.0), axis=1, keepdims=True)
    px1_row = jnp.transpose(p1 / z)
    tok = lam_row - coef_row * jnp.log(lam_row * px1_row + _EPS)
    return jnp.sum(tok)


def _loss_kernel(lp_ref, ld_ref, lv_ref,
                 xtp_ref, x0p_ref, x1p_ref, llp_ref,
                 xtd_ref, x0d_ref, x1d_ref, lld_ref,
                 xtv_ref, x0v_ref, x1v_ref, llv_ref,
                 tl_ref, out_ref):
    i = pl.program_id(0)

    @pl.when(i == 0)
    def _():
        out_ref[...] = jnp.zeros_like(out_ref)

    tl_row = tl_ref[0]
    acc = _attr_loss(lp_ref[...], xtp_ref[0], x0p_ref[0], x1p_ref[0],
                     llp_ref[0], tl_row, False)
    acc += _attr_loss(ld_ref[...], xtd_ref[0], x0d_ref[0], x1d_ref[0],
                      lld_ref[0], tl_row, True)
    acc += _attr_loss(lv_ref[...], xtv_ref[0], x0v_ref[0], x1v_ref[0],
                      llv_ref[0], tl_row, True)
    out_ref[...] += jnp.full((1, 1), acc)


def kernel(token_logits_pitch, token_logits_duration, token_logits_velocity,
           log_lambda_pitch, log_lambda_duration, log_lambda_velocity,
           x_t_pitch, x_t_duration, x_t_velocity,
           x_0_pitch, x_0_duration, x_0_velocity,
           x_1_pitch, x_1_duration, x_1_velocity,
           t):
    tl = 1.0 / jnp.clip(1.0 - t, 1e-8, None)
    tl_full = jnp.broadcast_to(tl[:, None], (_B, _S))

    def meta(x):
        return x.reshape(_NB, 1, _TB)

    args = [
        token_logits_pitch.reshape(_T, 128),
        token_logits_duration.reshape(_T, 512),
        token_logits_velocity.reshape(_T, 128),
        meta(x_t_pitch), meta(x_0_pitch), meta(x_1_pitch), meta(log_lambda_pitch),
        meta(x_t_duration), meta(x_0_duration), meta(x_1_duration), meta(log_lambda_duration),
        meta(x_t_velocity), meta(x_0_velocity), meta(x_1_velocity), meta(log_lambda_velocity),
        meta(tl_full),
    ]

    def logits_spec(v):
        return pl.BlockSpec((_TB, v), lambda i: (i, 0))

    meta_spec = pl.BlockSpec((1, 1, _TB), lambda i: (i, 0, 0))

    out = pl.pallas_call(
        _loss_kernel,
        grid=(_NB,),
        in_specs=[logits_spec(128), logits_spec(512), logits_spec(128)]
                 + [meta_spec] * 13,
        out_specs=pl.BlockSpec((1, 1), lambda i: (0, 0)),
        out_shape=jax.ShapeDtypeStruct((1, 1), jnp.float32),
    )(*args)
    return out[0, 0] / _T


# speedup vs baseline: 3.2233x; 1.0070x over previous
import jax
import jax.numpy as jnp
from jax import lax
from jax.experimental import pallas as pl

_B, _S = 4, 8192
_T = _B * _S
_TB = 4096
_NB = _T // _TB
_EPS = 1e-8
_NEG = float(jnp.finfo(jnp.float32).min)


def _softplus(x):
    return jnp.maximum(x, 0.0) + jnp.log1p(jnp.exp(-jnp.abs(x)))


def _attr_loss(l, xt_row, x0_row, x1_row, ll_row, tl_row, disallow_zero):
    TB, V = l.shape
    sm = (xt_row == x0_row) & (x1_row != xt_row)
    coef_row = jnp.where(sm, tl_row, 0.0)
    lam_row = _softplus(ll_row)

    xt = jnp.transpose(xt_row)
    x1 = jnp.transpose(x1_row)
    iota = lax.broadcasted_iota(jnp.int32, (TB, V), 1)
    maskm = iota == xt
    if disallow_zero:
        maskm = maskm | (iota == 0)
    lm = jnp.where(maskm, _NEG, l)
    m = jnp.max(lm, axis=1, keepdims=True)
    p = jnp.exp(lm - m)
    z = jnp.sum(p, axis=1, keepdims=True)
    p1 = jnp.sum(jnp.where(iota == x1, p, 0.0), axis=1, keepdims=True)
    px1_row = jnp.transpose(p1 / z)
    tok = lam_row - coef_row * jnp.log(lam_row * px1_row + _EPS)
    return jnp.sum(tok)


def _loss_kernel(lp_ref, ld_ref, lv_ref,
                 xtp_ref, x0p_ref, x1p_ref, llp_ref,
                 xtd_ref, x0d_ref, x1d_ref, lld_ref,
                 xtv_ref, x0v_ref, x1v_ref, llv_ref,
                 tl_ref, out_ref):
    i = pl.program_id(0)

    @pl.when(i == 0)
    def _():
        out_ref[...] = jnp.zeros_like(out_ref)

    tl_row = tl_ref[0]
    acc = _attr_loss(lp_ref[...], xtp_ref[0], x0p_ref[0], x1p_ref[0],
                     llp_ref[0], tl_row, False)
    acc += _attr_loss(ld_ref[...], xtd_ref[0], x0d_ref[0], x1d_ref[0],
                      lld_ref[0], tl_row, True)
    acc += _attr_loss(lv_ref[...], xtv_ref[0], x0v_ref[0], x1v_ref[0],
                      llv_ref[0], tl_row, True)
    out_ref[...] += jnp.full((1, 1), acc)


def kernel(token_logits_pitch, token_logits_duration, token_logits_velocity,
           log_lambda_pitch, log_lambda_duration, log_lambda_velocity,
           x_t_pitch, x_t_duration, x_t_velocity,
           x_0_pitch, x_0_duration, x_0_velocity,
           x_1_pitch, x_1_duration, x_1_velocity,
           t):
    tl = 1.0 / jnp.clip(1.0 - t, 1e-8, None)
    tl_full = jnp.broadcast_to(tl[:, None], (_B, _S))

    def meta(x):
        return x.reshape(_NB, 1, _TB)

    args = [
        token_logits_pitch.reshape(_T, 128),
        token_logits_duration.reshape(_T, 512),
        token_logits_velocity.reshape(_T, 128),
        meta(x_t_pitch), meta(x_0_pitch), meta(x_1_pitch), meta(log_lambda_pitch),
        meta(x_t_duration), meta(x_0_duration), meta(x_1_duration), meta(log_lambda_duration),
        meta(x_t_velocity), meta(x_0_velocity), meta(x_1_velocity), meta(log_lambda_velocity),
        meta(tl_full),
    ]

    def logits_spec(v):
        return pl.BlockSpec((_TB, v), lambda i: (i, 0))

    meta_spec = pl.BlockSpec((1, 1, _TB), lambda i: (i, 0, 0))

    out = pl.pallas_call(
        _loss_kernel,
        grid=(_NB,),
        in_specs=[logits_spec(128), logits_spec(512), logits_spec(128)]
                 + [meta_spec] * 13,
        out_specs=pl.BlockSpec((1, 1), lambda i: (0, 0)),
        out_shape=jax.ShapeDtypeStruct((1, 1), jnp.float32),
    )(*args)
    return out[0, 0] / _T
